# baseline (speedup 1.0000x reference)
import functools

import jax
import jax.numpy as jnp
from jax import lax
from jax.experimental import pallas as pl
from jax.experimental.pallas import tpu as pltpu
from jax.experimental.pallas import tpu_sc as plsc

BS, SEQ, IN_DIM, EMB = 4, 4096, 512, 1024
HID = EMB // 2
KK = max(1, int((SEQ - 2) * 0.3))
KPAD = 1280
PT = KPAD // 8
ROWS = BS * KPAD
NCH = SEQ // 16


def _sc_body(text_hbm, feat_hbm, out_hbm, lens_hbm,
             trow, idxa, idxb, rowsa, rowsb, nzv, sema, semb):
    c = lax.axis_index("c")
    sub = lax.axis_index("s")
    wid = 16 * c + sub
    b = wid // 8
    s = wid % 8
    lo = s * PT
    dstbase = b * KPAD + lo

    pltpu.sync_copy(text_hbm.at[pl.ds(b * SEQ, SEQ)], trow)

    lane = lax.iota(jnp.int32, 16)

    def abody(i, zt):
        t = trow[pl.ds(i * 16, 16)]
        return zt + plsc.all_reduce_population_count(t == 0)
    z_total = lax.fori_loop(0, NCH, abody, jnp.zeros((16,), jnp.int32))

    @pl.when(s == 0)
    def _():
        nzv[...] = z_total
        pltpu.sync_copy(nzv, lens_hbm.at[b])

    lo_vec = jnp.full((16,), lo, jnp.int32)
    src_base = jnp.full((16,), b * SEQ, jnp.int32) + lane

    def bbody(i, z_run):
        t = trow[pl.ds(i * 16, 16)]
        zb = t == 0
        zc = plsc.cumsum(zb.astype(jnp.int32))
        pos = i * 16 + lane
        rank = jnp.where(zb, z_run + zc,
                         z_total + pos + 1 - z_run - zc) - 1
        slot = rank - lo_vec
        ma = (slot >= 0) & (slot < 128)
        mb = (slot >= 128) & (slot < PT)
        srcv = src_base + i * 16
        plsc.store_scatter(idxa, [jnp.clip(slot, 0, 127)], srcv, mask=ma)
        plsc.store_scatter(idxb, [jnp.clip(slot - 128, 0, PT - 129)],
                           srcv, mask=mb)
        return z_run + plsc.all_reduce_population_count(zb)
    lax.fori_loop(0, NCH, bbody, jnp.zeros((16,), jnp.int32))

    ca = pltpu.async_copy(feat_hbm.at[idxa], rowsa, sema)
    cb = pltpu.async_copy(feat_hbm.at[idxb], rowsb, semb)
    ca.wait()
    pltpu.sync_copy(rowsa, out_hbm.at[pl.ds(dstbase, 128)])
    cb.wait()
    pltpu.sync_copy(rowsb, out_hbm.at[pl.ds(dstbase + 128, PT - 128)])


@functools.partial(
    pl.kernel,
    mesh=plsc.VectorSubcoreMesh(core_axis_name="c", subcore_axis_name="s"),
    compiler_params=pltpu.CompilerParams(needs_layout_passes=False),
    out_type=[
        jax.ShapeDtypeStruct((ROWS, IN_DIM), jnp.float32),
        jax.ShapeDtypeStruct((BS, 16), jnp.int32),
    ],
    scratch_types=[
        pltpu.VMEM((SEQ,), jnp.int32),
        pltpu.VMEM((128,), jnp.int32),
        pltpu.VMEM((PT - 128,), jnp.int32),
        pltpu.VMEM((128, IN_DIM), jnp.float32),
        pltpu.VMEM((PT - 128, IN_DIM), jnp.float32),
        pltpu.VMEM((16,), jnp.int32),
        pltpu.SemaphoreType.DMA,
        pltpu.SemaphoreType.DMA,
    ],
)
def _sc_gather(text_hbm, feat_hbm, out_hbm, lens_hbm, *scratch):
    _sc_body(text_hbm, feat_hbm, out_hbm, lens_hbm, *scratch)


def _tc_body(x_ref, wlt_ref, w0t_ref, w1t_ref, blin_ref, b0_ref, b1_ref,
             g0_ref, be0_ref, lens_ref, out_ref):
    x = x_ref[...]
    ssq = jnp.dot(x * x, jnp.ones((IN_DIM, 1), jnp.float32),
                  preferred_element_type=jnp.float32)
    xn = x / jnp.maximum(jnp.sqrt(ssq), 1e-6)
    xb = xn.astype(jnp.bfloat16)

    h = jnp.dot(xb, w0t_ref[...].astype(jnp.bfloat16),
                preferred_element_type=jnp.float32) + b0_ref[...]
    cidx = lax.broadcasted_iota(jnp.int32, (1, ROWS), 1)
    w_row = (cidx % KPAD < KK).astype(jnp.float32)
    denom = jnp.float32(BS * KK)
    mu = jnp.dot(w_row, h, preferred_element_type=jnp.float32) / denom
    m2 = jnp.dot(w_row, h * h, preferred_element_type=jnp.float32) / denom
    var = m2 - mu * mu
    hn = (h - mu) / jnp.sqrt(var + 1e-5) * g0_ref[...] + be0_ref[...]
    r = jnp.maximum(hn, 0.0)

    fused = (jnp.dot(r.astype(jnp.bfloat16), w1t_ref[...].astype(jnp.bfloat16),
                     preferred_element_type=jnp.float32)
             + jnp.dot(xb, wlt_ref[...].astype(jnp.bfloat16),
                       preferred_element_type=jnp.float32)
             + b1_ref[...] + blin_ref[...])

    neg = jnp.float32(-jnp.inf)
    for b in range(BS):
        plen = jnp.clip(SEQ - lens_ref[b, 0] - 2, 1, KK)
        seg = fused[b * KPAD:(b + 1) * KPAD, :]
        pm = lax.broadcasted_iota(jnp.int32, (KPAD, 1), 0) < plen
        out_ref[b, :] = jnp.max(jnp.where(pm, seg, neg), axis=0)


def _tc_dense(x, wlt, w0t, w1t, blin, b0, b1, g0, be0, lens):
    vspec = pl.BlockSpec(memory_space=pltpu.VMEM)
    return pl.pallas_call(
        _tc_body,
        out_shape=jax.ShapeDtypeStruct((BS, EMB), jnp.float32),
        in_specs=[vspec] * 9 + [pl.BlockSpec(memory_space=pltpu.SMEM)],
        out_specs=vspec,
        compiler_params=pltpu.CompilerParams(
            vmem_limit_bytes=128 * 1024 * 1024),
    )(x, wlt, w0t, w1t, blin, b0, b1, g0, be0, lens)


def kernel(features, text, atten, W_lin, b_lin, W0, b0, g0, be0, W1, b1):
    del atten
    feat_flat = features.reshape(BS * SEQ, IN_DIM)
    text_flat = text.reshape(BS * SEQ).astype(jnp.int32)

    feats_sc, lens = _sc_gather(text_flat, feat_flat)

    out = _tc_dense(
        feats_sc,
        W_lin.T, W0.T, W1.T,
        b_lin.reshape(1, EMB), b0.reshape(1, HID), b1.reshape(1, EMB),
        g0.reshape(1, HID), be0.reshape(1, HID),
        lens,
    )
    return out.astype(jnp.float32)
